# baseline (speedup 1.0000x reference)
import functools

import jax
import jax.numpy as jnp
from jax import lax
from jax.experimental import pallas as pl
from jax.experimental.pallas import tpu as pltpu
from jax.experimental.pallas import tpu_sc as plsc

BLOCK_ROWS = 1280
NUM_CORES = 2
NUM_SUBCORES = 16
NUM_WORKERS = NUM_CORES * NUM_SUBCORES


@functools.cache
def _make_gather(num_rows_total: int, dim: int):
    rpw = num_rows_total // NUM_WORKERS
    blocks = rpw // BLOCK_ROWS
    assert blocks % 2 == 0 and blocks >= 4
    mesh = plsc.VectorSubcoreMesh(core_axis_name="c", subcore_axis_name="s")

    @functools.partial(
        pl.kernel,
        mesh=mesh,
        out_type=jax.ShapeDtypeStruct((num_rows_total, dim), jnp.float32),
        scratch_types=[
            pltpu.VMEM((rpw,), jnp.int32),
            pltpu.VMEM((BLOCK_ROWS, dim), jnp.float32),
            pltpu.VMEM((BLOCK_ROWS, dim), jnp.float32),
            pltpu.SemaphoreType.DMA,
            pltpu.SemaphoreType.DMA,
            pltpu.SemaphoreType.DMA,
            pltpu.SemaphoreType.DMA,
        ],
        compiler_params=pltpu.CompilerParams(use_tc_tiling_on_sc=False),
    )
    def gather_kernel(
        idx_hbm, table_hbm, out_hbm, idx_v, rows_a, rows_b, sga, sgb, soa, sob
    ):
        wid = lax.axis_index("s") * NUM_CORES + lax.axis_index("c")
        r0 = wid * rpw
        pltpu.sync_copy(idx_hbm.at[pl.ds(r0, rpw)], idx_v)

        def fire(blk, rows_v, sem):
            pltpu.async_copy(
                table_hbm.at[pl.ds(r0 + blk * BLOCK_ROWS, BLOCK_ROWS)],
                rows_v,
                sem,
            )

        def drain_gathers(rows_v, sem):
            pltpu.make_async_copy(
                table_hbm.at[pl.ds(0, BLOCK_ROWS)], rows_v, sem
            ).wait()

        def copy_out(blk, rows_v, sem):
            return pltpu.async_copy(
                rows_v, out_hbm.at[pl.ds(r0 + blk * BLOCK_ROWS, BLOCK_ROWS)], sem
            )

        def drain_out(blk, rows_v, sem):
            pltpu.make_async_copy(
                rows_v, out_hbm.at[pl.ds(r0 + blk * BLOCK_ROWS, BLOCK_ROWS)], sem
            ).wait()

        fire(0, rows_a, sga)
        drain_gathers(rows_a, sga)
        copy_out(0, rows_a, soa)
        fire(1, rows_b, sgb)

        def body(ii, carry):
            b1 = 2 * ii + 1
            b2 = 2 * ii + 2
            drain_gathers(rows_b, sgb)
            drain_out(b2 - 2, rows_a, soa)
            fire(b2, rows_a, sga)
            copy_out(b1, rows_b, sob)
            drain_gathers(rows_a, sga)
            drain_out(b1, rows_b, sob)
            fire(b2 + 1, rows_b, sgb)
            copy_out(b2, rows_a, soa)
            return carry

        lax.fori_loop(0, blocks // 2 - 1, body, 0)

        drain_gathers(rows_b, sgb)
        drain_out(blocks - 2, rows_a, soa)
        copy_out(blocks - 1, rows_b, sob)
        drain_out(blocks - 1, rows_b, sob)

    return gather_kernel


def kernel(indices, weight):
    idx = jnp.arange(indices.size, dtype=jnp.int32)
    out = _make_gather(idx.shape[0], weight.shape[1])(idx, weight)
    return out.reshape(indices.shape + (weight.shape[1],))


# speedup vs baseline: 1.0012x; 1.0012x over previous
import functools

import jax
import jax.numpy as jnp
from jax import lax
from jax.experimental import pallas as pl
from jax.experimental.pallas import tpu as pltpu
from jax.experimental.pallas import tpu_sc as plsc

BLOCK_ROWS = 1280
NUM_CORES = 2
NUM_SUBCORES = 16
NUM_WORKERS = NUM_CORES * NUM_SUBCORES


@functools.cache
def _make_gather(num_rows_total: int, dim: int):
    rpw = num_rows_total // NUM_WORKERS
    assert rpw * NUM_WORKERS == num_rows_total
    blocks = rpw // BLOCK_ROWS
    assert blocks * BLOCK_ROWS == rpw and blocks % 2 == 0 and blocks >= 4
    mesh = plsc.VectorSubcoreMesh(core_axis_name="c", subcore_axis_name="s")

    @functools.partial(
        pl.kernel,
        mesh=mesh,
        out_type=jax.ShapeDtypeStruct((num_rows_total, dim), jnp.float32),
        scratch_types=[
            pltpu.VMEM((BLOCK_ROWS,), jnp.int32),
            pltpu.VMEM((BLOCK_ROWS,), jnp.int32),
            pltpu.VMEM((BLOCK_ROWS, dim), jnp.float32),
            pltpu.VMEM((BLOCK_ROWS, dim), jnp.float32),
            pltpu.SemaphoreType.DMA,
            pltpu.SemaphoreType.DMA,
            pltpu.SemaphoreType.DMA,
            pltpu.SemaphoreType.DMA,
            pltpu.SemaphoreType.DMA,
            pltpu.SemaphoreType.DMA,
        ],
        compiler_params=pltpu.CompilerParams(use_tc_tiling_on_sc=False),
    )
    def gather_kernel(
        idx_hbm, table_hbm, out_hbm,
        idx_a, idx_b, rows_a, rows_b,
        sia, sib, sga, sgb, soa, sob,
    ):
        wid = lax.axis_index("s") * NUM_CORES + lax.axis_index("c")
        r0 = wid * rpw

        idx_v = (idx_a, idx_b)
        rows_v = (rows_a, rows_b)
        si = (sia, sib)
        sg = (sga, sgb)
        so = (soa, sob)

        def fire_idx(b, p):
            pltpu.async_copy(
                idx_hbm.at[pl.ds(r0 + b * BLOCK_ROWS, BLOCK_ROWS)],
                idx_v[p],
                si[p],
            )

        def wait_idx(p):
            pltpu.make_async_copy(
                idx_hbm.at[pl.ds(0, BLOCK_ROWS)], idx_v[p], si[p]
            ).wait()

        def fire_gather(p):
            pltpu.async_copy(table_hbm.at[idx_v[p]], rows_v[p], sg[p])

        def wait_gather(p):
            pltpu.make_async_copy(
                table_hbm.at[idx_v[p]], rows_v[p], sg[p]
            ).wait()

        def out_slice(b):
            return out_hbm.at[pl.ds(r0 + b * BLOCK_ROWS, BLOCK_ROWS)]

        def fire_out(b, p):
            pltpu.async_copy(rows_v[p], out_slice(b), so[p])

        def wait_out(b, p):
            pltpu.make_async_copy(rows_v[p], out_slice(b), so[p]).wait()

        def body(b, p, q, fire_next=True, wait_prev_out=True,
                 has_next_idx=True):
            if fire_next:
                wait_idx(q)
                if wait_prev_out:
                    wait_out(b - 1, q)
                fire_gather(q)
            wait_gather(p)
            if has_next_idx:
                fire_idx(b + 2, p)
            fire_out(b, p)

        fire_idx(0, 0)
        fire_idx(1, 1)
        wait_idx(0)
        fire_gather(0)

        body(0, 0, 1, wait_prev_out=False)
        body(1, 1, 0)

        def loop_body(ii, carry):
            body(2 * ii, 0, 1)
            body(2 * ii + 1, 1, 0)
            return carry

        lax.fori_loop(1, blocks // 2 - 1, loop_body, 0)

        body(blocks - 2, 0, 1, has_next_idx=False)
        body(blocks - 1, 1, 0, fire_next=False, has_next_idx=False)

        wait_out(blocks - 2, 0)
        wait_out(blocks - 1, 1)

    return gather_kernel


def kernel(indices, weight):
    n_i, n_j = indices.shape
    dim = weight.shape[1]
    idx = indices.reshape(-1).astype(jnp.int32)
    out = _make_gather(idx.shape[0], dim)(idx, weight)
    return out.reshape(n_i, n_j, dim)
